# baseline (speedup 1.0000x reference)
import functools

import jax
import jax.numpy as jnp
from jax import lax
from jax.experimental import pallas as pl
from jax.experimental.pallas import tpu as pltpu
from jax.experimental.pallas import tpu_sc as plsc

VOCAB = 1000000
N_ENTS = 100
D = 64
CHUNK = 256
GROUPS = CHUNK // 16
NBLK = CHUNK // 128


def _body(idx_hbm, mask_hbm, words_hbm, ents_hbm, pad_hbm, out_hbm,
          idx_v, mask_v, pad_v, wrow_v, erow_v, half_v, buf_v, outb_v,
          sem_g, n_chunks, n_batch):
    info = plsc.get_sparse_core_info()
    nc = info.num_cores
    nw = info.num_subcores * nc
    wid = lax.axis_index("s") * nc + lax.axis_index("c")

    pltpu.sync_copy(pad_hbm, pad_v)
    pad16 = pad_v[...]
    iota16 = lax.iota(jnp.int32, 16)
    blocks_per_l = n_batch // CHUNK

    def run_chunk(chunk_id):
        base = chunk_id * CHUNK
        l_id = lax.div(chunk_id, blocks_per_l)
        b0 = lax.rem(chunk_id, blocks_per_l) * CHUNK
        pltpu.sync_copy(idx_hbm.at[pl.ds(base, CHUNK)], idx_v)
        pltpu.sync_copy(mask_hbm.at[pl.ds(base, CHUNK)], mask_v)

        for g in range(GROUPS):
            iv = idx_v[pl.ds(g * 16, 16)]
            mv = mask_v[pl.ds(g * 16, 16)]
            b = b0 + g * 16 + iota16
            ent = iv - VOCAB
            entc = jnp.where((ent < 0) | (ent >= N_ENTS), N_ENTS - 1, ent)
            erow = b * N_ENTS + entc
            wrow = jnp.where(iv >= VOCAB, pad16, iv)
            hsel = jnp.where(mv == 1, erow, wrow) & 1
            wrow_v[g // 8, pl.ds((g % 8) * 16, 16)] = (
                lax.shift_right_logical(wrow, 1))
            erow_v[g // 8, pl.ds((g % 8) * 16, 16)] = (
                lax.shift_right_logical(erow, 1))
            half_v[pl.ds(g * 16, 16)] = hsel * D

        copies = []
        for j in range(NBLK):
            copies.append(pltpu.async_copy(
                words_hbm.at[wrow_v.at[j]],
                buf_v.at[pl.ds(j * 128, 128)], sem_g))
            copies.append(pltpu.async_copy(
                ents_hbm.at[erow_v.at[j]],
                buf_v.at[pl.ds(CHUNK + j * 128, 128)], sem_g))
        for c in copies:
            c.wait()

        def select_group(g, carry):
            mv = mask_v[pl.ds(g * 16, 16)]
            hv = half_v[pl.ds(g * 16, 16)]
            for j in range(16):
                t = g * 16 + j
                row = t + mv[j] * CHUNK
                off = hv[j]
                for c in range(D // 16):
                    outb_v[t, pl.ds(c * 16, 16)] = (
                        buf_v[row, pl.ds(off + c * 16, 16)])
            return carry

        lax.fori_loop(0, GROUPS, select_group, 0)
        pltpu.sync_copy(outb_v, out_hbm.at[l_id, pl.ds(b0, CHUNK)])

    for k in range((n_chunks + nw - 1) // nw):
        chunk_id = wid + k * nw
        if (k + 1) * nw <= n_chunks:
            run_chunk(chunk_id)
        else:
            @pl.when(chunk_id < n_chunks)
            def _():
                run_chunk(chunk_id)


def kernel(caption_indices, entities_encoded, word_embedding, pad_token,
           caption_masks):
    B, L = caption_indices.shape
    n_tok = B * L
    n_chunks = n_tok // CHUNK
    idx = caption_indices.T.reshape(n_tok).astype(jnp.int32)
    msk = caption_masks.reshape(B, L).T.reshape(n_tok).astype(jnp.int32)
    words1 = word_embedding.reshape(VOCAB * D // 128, 128)
    ents1 = entities_encoded.reshape(B * N_ENTS * D // 128, 128)
    pad16 = jnp.full((16,), pad_token, jnp.int32)

    mesh = plsc.VectorSubcoreMesh(core_axis_name="c", subcore_axis_name="s")
    body = functools.partial(_body, n_chunks=n_chunks, n_batch=B)
    out = pl.kernel(
        body,
        out_type=jax.ShapeDtypeStruct((L, B, D), jnp.float32),
        mesh=mesh,
        compiler_params=pltpu.CompilerParams(use_tc_tiling_on_sc=False),
        scratch_types=[
            pltpu.VMEM((CHUNK,), jnp.int32),
            pltpu.VMEM((CHUNK,), jnp.int32),
            pltpu.VMEM((16,), jnp.int32),
            pltpu.VMEM((NBLK, 128), jnp.int32),
            pltpu.VMEM((NBLK, 128), jnp.int32),
            pltpu.VMEM((CHUNK,), jnp.int32),
            pltpu.VMEM((2 * CHUNK, 128), jnp.float32),
            pltpu.VMEM((CHUNK, D), jnp.float32),
            pltpu.SemaphoreType.DMA,
        ],
    )(idx, msk, words1, ents1, pad16)
    return out.transpose(1, 0, 2)


# speedup vs baseline: 1.0006x; 1.0006x over previous
import functools

import jax
import jax.numpy as jnp
from jax import lax
from jax.experimental import pallas as pl
from jax.experimental.pallas import tpu as pltpu
from jax.experimental.pallas import tpu_sc as plsc

VOCAB = 1000000
N_ENTS = 100
D = 64
CHUNK = 256
GROUPS = CHUNK // 16
NBLK = CHUNK // 128


def _body(idx_hbm, mask_hbm, words_hbm, ents_hbm, pad_hbm, out_hbm,
          idx_v, mask_v, pad_v, wrow_v, erow_v, half_v, buf_v, outb_v,
          sem_g, n_chunks, n_batch):
    info = plsc.get_sparse_core_info()
    nc = info.num_cores
    nw = info.num_subcores * nc
    wid = lax.axis_index("s") * nc + lax.axis_index("c")

    pltpu.sync_copy(pad_hbm, pad_v)
    pad16 = pad_v[...]
    iota16 = lax.iota(jnp.int32, 16)
    blocks_per_l = n_batch // CHUNK

    def run_chunk(chunk_id):
        base = chunk_id * CHUNK
        l_id = lax.div(chunk_id, blocks_per_l)
        b0 = lax.rem(chunk_id, blocks_per_l) * CHUNK
        pltpu.sync_copy(idx_hbm.at[pl.ds(base, CHUNK)], idx_v)
        pltpu.sync_copy(mask_hbm.at[pl.ds(base, CHUNK)], mask_v)

        for g in range(GROUPS):
            iv = idx_v[pl.ds(g * 16, 16)]
            mv = mask_v[pl.ds(g * 16, 16)]
            b = b0 + g * 16 + iota16
            ent = iv - VOCAB
            entc = jnp.where((ent < 0) | (ent >= N_ENTS), N_ENTS - 1, ent)
            erow = b * N_ENTS + entc
            wrow = jnp.where(iv >= VOCAB, pad16, iv)
            hsel = jnp.where(mv == 1, erow, wrow) & 1
            wrow_v[g // 8, pl.ds((g % 8) * 16, 16)] = (
                lax.shift_right_logical(wrow, 1))
            erow_v[g // 8, pl.ds((g % 8) * 16, 16)] = (
                lax.shift_right_logical(erow, 1))
            half_v[pl.ds(g * 16, 16)] = hsel * D

        copies = []
        for j in range(NBLK):
            copies.append(pltpu.async_copy(
                words_hbm.at[wrow_v.at[j]],
                buf_v.at[pl.ds(j * 128, 128)], sem_g))
            copies.append(pltpu.async_copy(
                ents_hbm.at[erow_v.at[j]],
                buf_v.at[pl.ds(CHUNK + j * 128, 128)], sem_g))
        for c in copies:
            c.wait()

        def select_group(g, carry):
            mv = mask_v[pl.ds(g * 16, 16)]
            hv = half_v[pl.ds(g * 16, 16)]
            for j in range(16):
                t = g * 16 + j
                row = t + mv[j] * CHUNK
                off = hv[j]
                for c in range(D // 16):
                    outb_v[t, pl.ds(c * 16, 16)] = (
                        buf_v[row, pl.ds(off + c * 16, 16)])
            return carry

        lax.fori_loop(0, GROUPS, select_group, 0)
        pltpu.sync_copy(outb_v, out_hbm.at[l_id, pl.ds(b0, CHUNK)])

    for k in range((n_chunks + nw - 1) // nw):
        chunk_id = wid + k * nw
        if (k + 1) * nw <= n_chunks:
            run_chunk(chunk_id)
        else:
            @pl.when(chunk_id < n_chunks)
            def _():
                run_chunk(chunk_id)


def kernel(caption_indices, entities_encoded, word_embedding, pad_token,
           caption_masks):
    B, L = caption_indices.shape
    n_tok = B * L
    n_chunks = n_tok // CHUNK
    idx = caption_indices.T.reshape(n_tok).astype(jnp.int32)
    msk = caption_masks.reshape(B, L).T.reshape(n_tok).astype(jnp.int32)
    words1 = jax.lax.optimization_barrier(
        word_embedding.reshape(VOCAB * D // 128, 128))
    ents1 = jax.lax.optimization_barrier(
        entities_encoded.reshape(B * N_ENTS * D // 128, 128))
    pad16 = jnp.full((16,), pad_token, jnp.int32)

    mesh = plsc.VectorSubcoreMesh(core_axis_name="c", subcore_axis_name="s")
    body = functools.partial(_body, n_chunks=n_chunks, n_batch=B)
    out = pl.kernel(
        body,
        out_type=jax.ShapeDtypeStruct((L, B, D), jnp.float32),
        mesh=mesh,
        compiler_params=pltpu.CompilerParams(use_tc_tiling_on_sc=False),
        scratch_types=[
            pltpu.VMEM((CHUNK,), jnp.int32),
            pltpu.VMEM((CHUNK,), jnp.int32),
            pltpu.VMEM((16,), jnp.int32),
            pltpu.VMEM((NBLK, 128), jnp.int32),
            pltpu.VMEM((NBLK, 128), jnp.int32),
            pltpu.VMEM((CHUNK,), jnp.int32),
            pltpu.VMEM((2 * CHUNK, 128), jnp.float32),
            pltpu.VMEM((CHUNK, D), jnp.float32),
            pltpu.SemaphoreType.DMA,
        ],
    )(idx, msk, words1, ents1, pad16)
    return out.transpose(1, 0, 2)


# speedup vs baseline: 1.0563x; 1.0557x over previous
import functools

import jax
import jax.numpy as jnp
from jax import lax
from jax.experimental import pallas as pl
from jax.experimental.pallas import tpu as pltpu
from jax.experimental.pallas import tpu_sc as plsc

VOCAB = 1000000
N_ENTS = 100
D = 64
CHUNK = 256
GROUPS = CHUNK // 16
NBLK = CHUNK // 128


def _body(idx_hbm, mask_hbm, words_hbm, ents_hbm, pad_hbm, out_hbm,
          idx_v, mask_v, pad_v, wrow_v, erow_v, buf_v, outb_v,
          sem_g, n_chunks, n_batch):
    info = plsc.get_sparse_core_info()
    nc = info.num_cores
    nw = info.num_subcores * nc
    wid = lax.axis_index("s") * nc + lax.axis_index("c")

    pltpu.sync_copy(pad_hbm, pad_v)
    pad16 = pad_v[...]
    iota16 = lax.iota(jnp.int32, 16)
    blocks_per_l = n_batch // CHUNK

    def run_chunk(chunk_id):
        base = chunk_id * CHUNK
        l_id = lax.div(chunk_id, blocks_per_l)
        b0 = lax.rem(chunk_id, blocks_per_l) * CHUNK
        pltpu.sync_copy(idx_hbm.at[pl.ds(base, CHUNK)], idx_v)
        pltpu.sync_copy(mask_hbm.at[pl.ds(base, CHUNK)], mask_v)

        for g in range(GROUPS):
            iv = idx_v[pl.ds(g * 16, 16)]
            mv = mask_v[pl.ds(g * 16, 16)]
            b = b0 + g * 16 + iota16
            ent = iv - VOCAB
            entc = jnp.where((ent < 0) | (ent >= N_ENTS), N_ENTS - 1, ent)
            erow = b * N_ENTS + entc
            wrow = jnp.where(iv >= VOCAB, pad16, iv)
            wrow_v[g // 8, pl.ds((g % 8) * 16, 16)] = wrow
            erow_v[g // 8, pl.ds((g % 8) * 16, 16)] = erow

        copies = []
        for j in range(NBLK):
            copies.append(pltpu.async_copy(
                words_hbm.at[wrow_v.at[j]],
                buf_v.at[pl.ds(j * 128, 128)], sem_g))
            copies.append(pltpu.async_copy(
                ents_hbm.at[erow_v.at[j]],
                buf_v.at[pl.ds(CHUNK + j * 128, 128)], sem_g))
        for c in copies:
            c.wait()

        def select_group(g, carry):
            mv = mask_v[pl.ds(g * 16, 16)]
            for j in range(16):
                t = g * 16 + j
                row = t + mv[j] * CHUNK
                for c in range(D // 16):
                    outb_v[t, pl.ds(c * 16, 16)] = (
                        buf_v[row, pl.ds(c * 16, 16)])
            return carry

        lax.fori_loop(0, GROUPS, select_group, 0)
        pltpu.sync_copy(outb_v, out_hbm.at[l_id, pl.ds(b0, CHUNK)])

    for k in range((n_chunks + nw - 1) // nw):
        chunk_id = wid + k * nw
        if (k + 1) * nw <= n_chunks:
            run_chunk(chunk_id)
        else:
            @pl.when(chunk_id < n_chunks)
            def _():
                run_chunk(chunk_id)


def kernel(caption_indices, entities_encoded, word_embedding, pad_token,
           caption_masks):
    B, L = caption_indices.shape
    n_tok = B * L
    n_chunks = n_tok // CHUNK
    idx = caption_indices.T.reshape(n_tok).astype(jnp.int32)
    msk = caption_masks.reshape(B, L).T.reshape(n_tok).astype(jnp.int32)
    words1 = jnp.pad(word_embedding, ((0, 0), (0, 128 - D)))
    ents1 = jnp.pad(entities_encoded.reshape(B * N_ENTS, D),
                    ((0, 0), (0, 128 - D)))
    pad16 = jnp.full((16,), pad_token, jnp.int32)

    mesh = plsc.VectorSubcoreMesh(core_axis_name="c", subcore_axis_name="s")
    body = functools.partial(_body, n_chunks=n_chunks, n_batch=B)
    out = pl.kernel(
        body,
        out_type=jax.ShapeDtypeStruct((L, B, D), jnp.float32),
        mesh=mesh,
        compiler_params=pltpu.CompilerParams(use_tc_tiling_on_sc=False),
        scratch_types=[
            pltpu.VMEM((CHUNK,), jnp.int32),
            pltpu.VMEM((CHUNK,), jnp.int32),
            pltpu.VMEM((16,), jnp.int32),
            pltpu.VMEM((NBLK, 128), jnp.int32),
            pltpu.VMEM((NBLK, 128), jnp.int32),
            pltpu.VMEM((2 * CHUNK, 128), jnp.float32),
            pltpu.VMEM((CHUNK, D), jnp.float32),
            pltpu.SemaphoreType.DMA,
        ],
    )(idx, msk, words1, ents1, pad16)
    return out.transpose(1, 0, 2)
